# baseline (speedup 1.0000x reference)
import functools

import jax
import jax.numpy as jnp
from jax import lax
from jax.experimental import pallas as pl
from jax.experimental.pallas import tpu as pltpu
from jax.experimental.pallas import tpu_sc as plsc


_BS = 1024
_R = 64


def _tc_add_body(x_ref, emb_ref, out_ref):
    out_ref[...] = x_ref[...] + emb_ref[...][None, :, :]


def _tc_kernel(x, emb_table):
    B, S, D = x.shape
    bs = _BS
    while S % bs:
        bs //= 2
    return pl.pallas_call(
        _tc_add_body,
        grid=(S // bs,),
        in_specs=[
            pl.BlockSpec((B, bs, D), lambda i: (0, i, 0)),
            pl.BlockSpec((bs, D), lambda i: (i, 0)),
        ],
        out_specs=pl.BlockSpec((B, bs, D), lambda i: (0, i, 0)),
        out_shape=jax.ShapeDtypeStruct((B, S, D), x.dtype),
    )(x, emb_table)


def _sc_kernel(x, emb_table):
    B, S, D = x.shape
    info = plsc.get_sparse_core_info()
    nw = info.num_cores * info.num_subcores
    rows_per_w = S // nw
    n_chunks = rows_per_w // _R
    mesh = plsc.VectorSubcoreMesh(core_axis_name="c", subcore_axis_name="s")

    @functools.partial(
        pl.kernel,
        mesh=mesh,
        out_type=jax.ShapeDtypeStruct((B, S, D), jnp.float32),
        scratch_types=[
            pltpu.VMEM((_R, D), jnp.float32),
            pltpu.VMEM((_R, D), jnp.float32),
        ],
    )
    def k(x_hbm, emb_hbm, out_hbm, emb_v, x_v):
        wid = lax.axis_index("s") * info.num_cores + lax.axis_index("c")
        base0 = wid * rows_per_w

        def chunk_body(c, carry):
            base = base0 + c * _R
            pltpu.sync_copy(emb_hbm.at[pl.ds(base, _R)], emb_v)
            for b in range(B):
                pltpu.sync_copy(x_hbm.at[b, pl.ds(base, _R)], x_v)

                def row_body(r, carry2):
                    for kk in range(D // info.num_lanes):
                        sl = pl.ds(kk * info.num_lanes, info.num_lanes)
                        x_v[r, sl] = x_v[r, sl] + emb_v[r, sl]
                    return carry2

                lax.fori_loop(0, _R, row_body, 0)
                pltpu.sync_copy(x_v, out_hbm.at[b, pl.ds(base, _R)])
            return carry

        lax.fori_loop(0, n_chunks, chunk_body, 0)

    return k(x, emb_table)


def kernel(x, emb_table):
    return _tc_kernel(x, emb_table)
